# baseline (speedup 1.0000x reference)
import functools

import jax
import jax.numpy as jnp
from jax import lax
from jax.experimental import pallas as pl
from jax.experimental.pallas import tpu as pltpu

_NUM_BINS = 64
_N = 1048576
_C = 4096


def _tc_body(x_ref, o_ref):
    x = x_ref[...].reshape(1, _C)
    t = x * jnp.float32(_NUM_BINS)
    fi = t.astype(jnp.int32)
    exact = fi.astype(jnp.float32) == t
    binid = jnp.maximum(jnp.where(exact, fi - 1, fi), 0)
    rows = lax.broadcasted_iota(jnp.int32, (_NUM_BINS, _C), 0)
    o_ref[...] = binid == rows


def kernel(sm_vector):
    return pl.pallas_call(
        _tc_body,
        grid=(_N // _C,),
        in_specs=[pl.BlockSpec((_C,), lambda j: (j,))],
        out_specs=pl.BlockSpec((_NUM_BINS, _C), lambda j: (0, j)),
        out_shape=jax.ShapeDtypeStruct((_NUM_BINS, _N), jnp.bool_),
        compiler_params=pltpu.CompilerParams(
            dimension_semantics=("arbitrary",),
        ),
    )(sm_vector)


# speedup vs baseline: 1.4263x; 1.4263x over previous
import jax
import jax.numpy as jnp
from jax import lax
from jax.experimental import pallas as pl
from jax.experimental.pallas import tpu as pltpu

_NUM_BINS = 64
_N = 1048576
_C = 8192
_W = 8 * _C
_STEPS = _N // _W


def _tc_body(x_ref, o_ref):
    x = x_ref[...]
    t = x * jnp.float32(_NUM_BINS)
    fi = t.astype(jnp.int32)
    exact = fi.astype(jnp.float32) == t
    binid = jnp.maximum(jnp.where(exact, fi - 1, fi), 0)
    rows = lax.broadcasted_iota(jnp.int32, (_NUM_BINS, _C), 0)
    for k in range(8):
        bk = lax.broadcast_in_dim(binid[k, :], (_NUM_BINS, _C), (1,))
        o_ref[:, pl.ds(k * _C, _C)] = bk == rows


def kernel(sm_vector):
    x2d = sm_vector.reshape(_N // _C, _C)
    return pl.pallas_call(
        _tc_body,
        grid=(_STEPS,),
        in_specs=[pl.BlockSpec((8, _C), lambda j: (j, 0))],
        out_specs=pl.BlockSpec((_NUM_BINS, _W), lambda j: (0, j)),
        out_shape=jax.ShapeDtypeStruct((_NUM_BINS, _N), jnp.bool_),
        compiler_params=pltpu.CompilerParams(
            dimension_semantics=("arbitrary",),
        ),
    )(x2d)


# speedup vs baseline: 5.3392x; 3.7434x over previous
import jax
import jax.numpy as jnp
from jax import lax
from jax.experimental import pallas as pl
from jax.experimental.pallas import tpu as pltpu

_NUM_BINS = 64
_N = 1048576
_R = 128
_C = _N // _R


def _bin_body(x_ref, o_ref):
    x = x_ref[...]
    t = x * jnp.float32(_NUM_BINS)
    fi = t.astype(jnp.int32)
    exact = fi.astype(jnp.float32) == t
    binid = jnp.maximum(jnp.where(exact, fi - 1, fi), 0)
    o_ref[...] = binid.astype(jnp.int8)


def kernel(sm_vector):
    x2d = sm_vector.reshape(_R, _C)
    binid = pl.pallas_call(
        _bin_body,
        out_shape=jax.ShapeDtypeStruct((_R, _C), jnp.int8),
    )(x2d)
    row_ids = lax.broadcasted_iota(jnp.int8, (_NUM_BINS, 1), 0)
    return binid.reshape(1, _N) == row_ids
